# baseline (speedup 1.0000x reference)
import functools

import jax
import jax.numpy as jnp
from jax import lax
from jax.experimental import pallas as pl
from jax.experimental.pallas import tpu as pltpu
from jax.experimental.pallas import tpu_sc as plsc

_B, _H, _W, _C = 2, 384, 384, 96
_D = 128
_NP = _B * _H * _W
_NW = 32
_K = 64
_CPR = _W // _K
_CPW = (_B * _H // _NW) * _CPR
_G = _K // 16
_NC = _C // 16
_FRW = _CPW * _K // _D


def _warp_kernel(table, fyr, fxr, out_hbm, *refs):
  (fybuf, fxbuf, idx0, idx1, ax0, ay0, ax1, ay1,
   gb0, gb1, ob0, ob1, sem_g, sem_g2, sem_g3, sem_g4, sem_o) = refs
  gsems = (sem_g, sem_g2, sem_g3, sem_g4)
  idxs = (idx0, idx1)
  bufs = (gb0, gb1)
  axr = (ax0, ax1)
  ayr = (ay0, ay1)
  obs = (ob0, ob1)

  wid = lax.axis_index("s") * 2 + lax.axis_index("c")
  c0 = wid * _CPW

  pltpu.sync_copy(fyr.at[pl.ds(wid * _FRW, _FRW), :], fybuf)
  pltpu.sync_copy(fxr.at[pl.ds(wid * _FRW, _FRW), :], fxbuf)

  def setup(t, b):
    cid = c0 + t
    r = cid // _CPR
    xbase = (cid % _CPR) * _K
    y = r % _H
    boff = (r // _H) * (_H * _W)
    lrow = (t * _K) // _D
    foff = (t * _K) % _D
    yf = y.astype(jnp.float32)
    for g in range(_G):
      sl = pl.ds(g * 16, 16)
      fsl = pl.ds(foff + g * 16, 16)
      qx = (xbase + g * 16 + lax.iota(jnp.int32, 16)).astype(jnp.float32) \
          - fxbuf[lrow, fsl]
      qy = yf - fybuf[lrow, fsl]
      qxc = jnp.minimum(jnp.maximum(qx, -1.0), jnp.float32(_W))
      qyc = jnp.minimum(jnp.maximum(qy, -1.0), jnp.float32(_H))
      fxi = jnp.minimum(jnp.maximum(qxc.astype(jnp.int32), 0), _W - 2)
      fyi = jnp.minimum(jnp.maximum(qyc.astype(jnp.int32), 0), _H - 2)
      axw = jnp.minimum(jnp.maximum(qx - fxi.astype(jnp.float32), 0.0), 1.0)
      ayw = jnp.minimum(jnp.maximum(qy - fyi.astype(jnp.float32), 0.0), 1.0)
      base = boff + fyi * _W + fxi
      idxs[b][sl] = base
      idxs[b][pl.ds(_K + g * 16, 16)] = base + 1
      idxs[b][pl.ds(2 * _K + g * 16, 16)] = base + _W
      idxs[b][pl.ds(3 * _K + g * 16, 16)] = base + _W + 1
      axr[b][sl] = axw
      ayr[b][sl] = ayw
    for i in range(4):
      pltpu.async_copy(table.at[idxs[b].at[pl.ds(i * _K, _K)]],
                       bufs[b].at[pl.ds(i * _K, _K), :], gsems[i])

  setup(0, 0)

  @pl.loop(0, _CPW, step=2)
  def _chunk_pair(t2):
    for b in (0, 1):
      t = t2 + b
      for i in range(4):
        pltpu.make_async_copy(table.at[idxs[b].at[pl.ds(i * _K, _K)]],
                              bufs[b].at[pl.ds(i * _K, _K), :], gsems[i]).wait()

      @pl.when(t + 1 < _CPW)
      def _():
        setup(t + 1, 1 - b)

      @pl.when(t >= 2)
      def _():
        pltpu.make_async_copy(
            obs[b], out_hbm.at[0, 0, pl.ds(0, _K), :], sem_o).wait()

      gb = bufs[b]
      ob = obs[b]

      @pl.loop(0, _G)
      def _blend_group(g):
        axg = axr[b][pl.ds(g * 16, 16)]
        ayg = ayr[b][pl.ds(g * 16, 16)]
        for l in range(16):
          i = g * 16 + l
          axs = jnp.broadcast_to(axg[l], (16,))
          ays = jnp.broadcast_to(ayg[l], (16,))
          for c in range(_NC):
            csl = pl.ds(c * 16, 16)
            vtl = gb[i, csl]
            vtr = gb[_K + i, csl]
            vbl = gb[2 * _K + i, csl]
            vbr = gb[3 * _K + i, csl]
            top = axs * (vtr - vtl) + vtl
            bot = axs * (vbr - vbl) + vbl
            ob[i, csl] = ays * (bot - top) + top

      cid = c0 + t
      r = cid // _CPR
      pltpu.async_copy(
          ob,
          out_hbm.at[r // _H, r % _H, pl.ds((cid % _CPR) * _K, _K), :],
          sem_o)

  for _ in range(2):
    pltpu.make_async_copy(
        obs[0], out_hbm.at[0, 0, pl.ds(0, _K), :], sem_o).wait()


_warp = functools.partial(
    pl.kernel,
    out_type=jax.ShapeDtypeStruct((_B, _H, _W, _C), jnp.float32),
    mesh=plsc.VectorSubcoreMesh(core_axis_name="c", subcore_axis_name="s"),
    compiler_params=pltpu.CompilerParams(
        needs_layout_passes=False, use_tc_tiling_on_sc=True),
    scratch_types=(
        [pltpu.VMEM((_FRW, _D), jnp.float32)] * 2
        + [pltpu.VMEM((4 * _K,), jnp.int32)] * 2
        + [pltpu.VMEM((_K,), jnp.float32)] * 4
        + [pltpu.VMEM((4 * _K, _D), jnp.float32)] * 2
        + [pltpu.VMEM((_K, _C), jnp.float32)] * 2
        + [pltpu.SemaphoreType.DMA] * 5
    ),
)(_warp_kernel)


def kernel(image, flow):
  table = jnp.pad(image, ((0, 0), (0, 0), (0, 0), (0, _D - _C)))
  table = table.reshape(_NP, _D)
  fyr = flow[..., 0].reshape(_NP // _D, _D)
  fxr = flow[..., 1].reshape(_NP // _D, _D)
  return _warp(table, fyr, fxr)


# speedup vs baseline: 1.0225x; 1.0225x over previous
import functools

import jax
import jax.numpy as jnp
from jax import lax
from jax.experimental import pallas as pl
from jax.experimental.pallas import tpu as pltpu
from jax.experimental.pallas import tpu_sc as plsc

_B, _H, _W, _C = 2, 384, 384, 96
_D = 128
_NP = _B * _H * _W
_NW = 32
_K = 64
_CPR = _W // _K
_CPW = (_B * _H // _NW) * _CPR
_G = _K // 16
_NC = _C // 16
_FRW = _CPW * _K // _D


def _warp_kernel(table, fyr, fxr, out_hbm, *refs):
  (fybuf, fxbuf, idx0, idx1, ax0, ay0, ax1, ay1,
   gb0, gb1, ob0, ob1, sem_g, sem_g2, sem_o) = refs
  idxs = (idx0, idx1)
  bufs = (gb0, gb1)
  axr = (ax0, ax1)
  ayr = (ay0, ay1)
  obs = (ob0, ob1)

  wid = lax.axis_index("s") * 2 + lax.axis_index("c")
  c0 = wid * _CPW

  pltpu.sync_copy(fyr.at[pl.ds(wid * _FRW, _FRW), :], fybuf)
  pltpu.sync_copy(fxr.at[pl.ds(wid * _FRW, _FRW), :], fxbuf)

  def setup(t, b):
    cid = c0 + t
    r = cid // _CPR
    xbase = (cid % _CPR) * _K
    y = r % _H
    boff = (r // _H) * (_H * _W)
    lrow = (t * _K) // _D
    foff = (t * _K) % _D
    yf = y.astype(jnp.float32)
    for g in range(_G):
      sl = pl.ds(g * 16, 16)
      fsl = pl.ds(foff + g * 16, 16)
      qx = (xbase + g * 16 + lax.iota(jnp.int32, 16)).astype(jnp.float32) \
          - fxbuf[lrow, fsl]
      qy = yf - fybuf[lrow, fsl]
      qxc = jnp.minimum(jnp.maximum(qx, -1.0), jnp.float32(_W))
      qyc = jnp.minimum(jnp.maximum(qy, -1.0), jnp.float32(_H))
      fxi = jnp.minimum(jnp.maximum(qxc.astype(jnp.int32), 0), _W - 2)
      fyi = jnp.minimum(jnp.maximum(qyc.astype(jnp.int32), 0), _H - 2)
      axw = jnp.minimum(jnp.maximum(qx - fxi.astype(jnp.float32), 0.0), 1.0)
      ayw = jnp.minimum(jnp.maximum(qy - fyi.astype(jnp.float32), 0.0), 1.0)
      base = boff + fyi * _W + fxi
      idxs[b][sl] = base
      idxs[b][pl.ds(_K + g * 16, 16)] = base + 1
      idxs[b][pl.ds(2 * _K + g * 16, 16)] = base + _W
      idxs[b][pl.ds(3 * _K + g * 16, 16)] = base + _W + 1
      axr[b][sl] = axw
      ayr[b][sl] = ayw
    pltpu.async_copy(table.at[idxs[b].at[pl.ds(0, 2 * _K)]],
                     bufs[b].at[pl.ds(0, 2 * _K), :], sem_g)
    pltpu.async_copy(table.at[idxs[b].at[pl.ds(2 * _K, 2 * _K)]],
                     bufs[b].at[pl.ds(2 * _K, 2 * _K), :], sem_g2)

  setup(0, 0)

  @pl.loop(0, _CPW, step=2)
  def _chunk_pair(t2):
    for b in (0, 1):
      t = t2 + b
      pltpu.make_async_copy(table.at[idxs[b].at[pl.ds(0, 2 * _K)]],
                            bufs[b].at[pl.ds(0, 2 * _K), :], sem_g).wait()
      pltpu.make_async_copy(table.at[idxs[b].at[pl.ds(2 * _K, 2 * _K)]],
                            bufs[b].at[pl.ds(2 * _K, 2 * _K), :], sem_g2).wait()

      @pl.when(t + 1 < _CPW)
      def _():
        setup(t + 1, 1 - b)

      @pl.when(t >= 2)
      def _():
        pltpu.make_async_copy(
            obs[b], out_hbm.at[0, 0, pl.ds(0, _K), :], sem_o).wait()

      gb = bufs[b]
      ob = obs[b]

      @pl.loop(0, _G)
      def _blend_group(g):
        axg = axr[b][pl.ds(g * 16, 16)]
        ayg = ayr[b][pl.ds(g * 16, 16)]
        for l in range(16):
          i = g * 16 + l
          axs = jnp.broadcast_to(axg[l], (16,))
          ays = jnp.broadcast_to(ayg[l], (16,))
          for c in range(_NC):
            csl = pl.ds(c * 16, 16)
            vtl = gb[i, csl]
            vtr = gb[_K + i, csl]
            vbl = gb[2 * _K + i, csl]
            vbr = gb[3 * _K + i, csl]
            top = axs * (vtr - vtl) + vtl
            bot = axs * (vbr - vbl) + vbl
            ob[i, csl] = ays * (bot - top) + top

      cid = c0 + t
      r = cid // _CPR
      pltpu.async_copy(
          ob,
          out_hbm.at[r // _H, r % _H, pl.ds((cid % _CPR) * _K, _K), :],
          sem_o)

  for _ in range(2):
    pltpu.make_async_copy(
        obs[0], out_hbm.at[0, 0, pl.ds(0, _K), :], sem_o).wait()


_warp = functools.partial(
    pl.kernel,
    out_type=jax.ShapeDtypeStruct((_B, _H, _W, _C), jnp.float32),
    mesh=plsc.VectorSubcoreMesh(core_axis_name="c", subcore_axis_name="s"),
    compiler_params=pltpu.CompilerParams(
        needs_layout_passes=False, use_tc_tiling_on_sc=True),
    scratch_types=(
        [pltpu.VMEM((_FRW, _D), jnp.float32)] * 2
        + [pltpu.VMEM((4 * _K,), jnp.int32)] * 2
        + [pltpu.VMEM((_K,), jnp.float32)] * 4
        + [pltpu.VMEM((4 * _K, _D), jnp.float32)] * 2
        + [pltpu.VMEM((_K, _C), jnp.float32)] * 2
        + [pltpu.SemaphoreType.DMA] * 3
    ),
)(_warp_kernel)


def kernel(image, flow):
  table = jnp.pad(image, ((0, 0), (0, 0), (0, 0), (0, _D - _C)))
  table = table.reshape(_NP, _D)
  fyr = flow[..., 0].reshape(_NP // _D, _D)
  fxr = flow[..., 1].reshape(_NP // _D, _D)
  return _warp(table, fyr, fxr)
